# baseline (speedup 1.0000x reference)
import functools

import jax
import jax.numpy as jnp
from jax import lax
from jax.experimental import pallas as pl
from jax.experimental.pallas import tpu as pltpu
from jax.experimental.pallas import tpu_sc as plsc

N = 10000
E = 320000
H = 128
NC = 2
NS = 16
NW = NC * NS
K = 128
EPT = 10240
BATCHES = EPT // K
E_PAD = NW * EPT
GARBAGE_ROW = N
AI_ROWS = N + 16
RPT = 624


def _mm_body(x_ref, wt_ref, b_ref, o_ref):
    acc = jnp.dot(x_ref[...], wt_ref[...], preferred_element_type=jnp.float32)
    o_ref[...] = jnp.maximum(acc + b_ref[...], 0.0)


def _matmul_relu(x2, wt, b2):
    return pl.pallas_call(
        _mm_body,
        grid=(20,),
        in_specs=[
            pl.BlockSpec((1000, H), lambda i: (i, 0)),
            pl.BlockSpec((H, H), lambda i: (0, 0)),
            pl.BlockSpec((1, H), lambda i: (0, 0)),
        ],
        out_specs=pl.BlockSpec((1000, H), lambda i: (i, 0)),
        out_shape=jax.ShapeDtypeStruct((2 * N, H), jnp.float32),
    )(x2, wt, b2)


def _sc_body(idx_hbm, table_hbm, out_hbm,
             crb0, crb1, crb2, crb3, g0, g1, ai_sh,
             semg0, semg1, semi0, semi1, semi2, semi3, ss0, ss1):
    cid = lax.axis_index("c")
    sid = lax.axis_index("s")
    wid = cid * NS + sid

    zero16 = jnp.zeros((16,), jnp.float32)

    def _zrow(r, c):
        for j in range(H // 16):
            g0[r, pl.ds(j * 16, 16)] = zero16
        return c

    lax.fori_loop(0, K, _zrow, 0)
    for k in range(RPT // K):
        pltpu.sync_copy(g0, ai_sh.at[pl.ds(sid * RPT + k * K, K)])
    _rem = RPT - (RPT // K) * K
    pltpu.sync_copy(g0.at[pl.ds(0, _rem)],
                    ai_sh.at[pl.ds(sid * RPT + (RPT // K) * K, _rem)])

    @pl.when(sid == NS - 1)
    def _zero_tail():
        pltpu.sync_copy(g0.at[pl.ds(0, AI_ROWS - NS * RPT)],
                        ai_sh.at[pl.ds(NS * RPT, AI_ROWS - NS * RPT)])

    plsc.subcore_barrier()

    crbs = (crb0, crb1, crb2, crb3)
    gs = (g0, g1)
    semis = (semi0, semi1, semi2, semi3)
    semgs = (semg0, semg1)
    sss = (ss0, ss1)

    def _gather(b_mod4, b_mod2, bt):
        del bt
        pltpu.async_copy(table_hbm.at[crbs[b_mod4].at[0]],
                         gs[b_mod2], semgs[b_mod2])

    def _scatter(b_mod4, b_mod2):
        pltpu.async_copy(gs[b_mod2], ai_sh.at[crbs[b_mod4].at[1]],
                         sss[b_mod2], add=True)

    def _wait_scatter(b_mod2):
        pltpu.make_async_copy(gs[b_mod2], ai_sh.at[crbs[0].at[1]],
                              sss[b_mod2]).wait()

    def _wait_gather(b_mod2):
        pltpu.make_async_copy(table_hbm.at[crbs[0].at[0]],
                              gs[b_mod2], semgs[b_mod2]).wait()

    def _load_idx(b_mod4, bt):
        pltpu.async_copy(idx_hbm.at[wid, bt], crbs[b_mod4], semis[b_mod4])

    def _wait_idx(b_mod4, bt):
        pltpu.make_async_copy(idx_hbm.at[wid, bt], crbs[b_mod4],
                              semis[b_mod4]).wait()

    pltpu.sync_copy(idx_hbm.at[wid, 0], crb0)
    pltpu.sync_copy(idx_hbm.at[wid, 1], crb1)
    _gather(0, 0, 0)
    _load_idx(2, 2)
    _gather(1, 1, 1)
    _load_idx(3, 3)
    _wait_gather(0)
    _scatter(0, 0)

    def _edge_body(t, c):
        b0 = 4 * t + 2
        for k in range(4):
            ci = (2 + k) % 4
            gi = k % 2
            _wait_scatter(gi)
            _wait_idx(ci, b0 + k)
            _gather(ci, gi, b0 + k)
            _load_idx((ci + 2) % 4, b0 + k + 2)
            _wait_gather(1 - gi)
            _scatter((ci + 3) % 4, 1 - gi)
        return c

    lax.fori_loop(0, BATCHES // 4, _edge_body, 0)

    _wait_gather(1)
    _scatter(1, 1)
    _wait_scatter(0)
    _wait_scatter(1)
    _wait_idx(2, BATCHES + 2)
    _wait_idx(3, BATCHES + 3)

    plsc.subcore_barrier()
    pltpu.sync_copy(ai_sh.at[pl.ds(sid * RPT, RPT)],
                    out_hbm.at[pl.ds(cid * N + sid * RPT, RPT)])

    @pl.when(sid == NS - 1)
    def _write_tail():
        pltpu.sync_copy(ai_sh.at[pl.ds(NS * RPT, N - NS * RPT)],
                        out_hbm.at[pl.ds(cid * N + NS * RPT, N - NS * RPT)])


@functools.cache
def _sc_scatter():
    return pl.kernel(
        _sc_body,
        out_type=jax.ShapeDtypeStruct((NC * N, H), jnp.float32),
        mesh=plsc.VectorSubcoreMesh(core_axis_name="c", subcore_axis_name="s"),
        scratch_types=[
            pltpu.VMEM((2, K), jnp.int32),
            pltpu.VMEM((2, K), jnp.int32),
            pltpu.VMEM((2, K), jnp.int32),
            pltpu.VMEM((2, K), jnp.int32),
            pltpu.VMEM((K, H), jnp.float32),
            pltpu.VMEM((K, H), jnp.float32),
            pltpu.VMEM_SHARED((AI_ROWS, H), jnp.float32),
            pltpu.SemaphoreType.DMA,
            pltpu.SemaphoreType.DMA,
            pltpu.SemaphoreType.DMA,
            pltpu.SemaphoreType.DMA,
            pltpu.SemaphoreType.DMA,
            pltpu.SemaphoreType.DMA,
            pltpu.SemaphoreType.DMA,
            pltpu.SemaphoreType.DMA,
        ],
    )


def _fin_body(sir_ref, ai_ref, x4_ref, lnw_ref, lnb_ref, o_ref):
    s = sir_ref[0]
    i = sir_ref[1]
    ai = ai_ref[0] + ai_ref[1]
    x4 = x4_ref[...]
    beta = x4[:, 0:1]
    gamma = x4[:, 1:2]
    ds = -beta * (ai * s)
    di = -ds - gamma * i
    dr = gamma * i
    w = lnw_ref[...]
    b = lnb_ref[...]

    def _ln(v):
        m = jnp.mean(v, axis=-1, keepdims=True)
        cvar = v - m
        var = jnp.mean(cvar * cvar, axis=-1, keepdims=True)
        return cvar * lax.rsqrt(var + 1e-5) * w + b

    o_ref[0] = _ln(ds)
    o_ref[1] = _ln(di)
    o_ref[2] = _ln(dr)
    o_ref[3] = x4


def _finalize(sir2, ai2, x4, lnw2, lnb2):
    return pl.pallas_call(
        _fin_body,
        grid=(10,),
        in_specs=[
            pl.BlockSpec((2, 1000, H), lambda j: (0, j, 0)),
            pl.BlockSpec((2, 1000, H), lambda j: (0, j, 0)),
            pl.BlockSpec((1000, H), lambda j: (j, 0)),
            pl.BlockSpec((1, H), lambda j: (0, 0)),
            pl.BlockSpec((1, H), lambda j: (0, 0)),
        ],
        out_specs=pl.BlockSpec((4, 1000, H), lambda j: (0, j, 0)),
        out_shape=jax.ShapeDtypeStruct((4, N, H), jnp.float32),
    )(sir2, ai2, x4, lnw2, lnb2)


def kernel(t, x, edge_index, W, b, ln_w, ln_b):
    del t
    x2 = x[: 2 * N]
    x4 = x[3 * N:]

    sir = _matmul_relu(x2, W.T, b.reshape(1, H))

    n_pad = E_PAD - E
    rows = jnp.concatenate(
        [edge_index[0], jnp.full((n_pad,), GARBAGE_ROW, jnp.int32)])
    cols = jnp.concatenate(
        [edge_index[1] + jnp.int32(N), jnp.zeros((n_pad,), jnp.int32)])
    idx = jnp.stack([cols.reshape(NW, BATCHES, K),
                     rows.reshape(NW, BATCHES, K)], axis=2)
    dummy = jnp.stack([jnp.zeros((NW, 4, K), jnp.int32),
                       jnp.full((NW, 4, K), GARBAGE_ROW, jnp.int32)], axis=2)
    idx = jnp.concatenate([idx, dummy], axis=1)

    ai_partials = _sc_scatter()(idx, sir)

    out = _finalize(sir.reshape(2, N, H), ai_partials.reshape(2, N, H),
                    x4, ln_w.reshape(1, H), ln_b.reshape(1, H))
    return out.reshape(4 * N, H)


# speedup vs baseline: 1.5174x; 1.5174x over previous
import functools

import jax
import jax.numpy as jnp
from jax import lax
from jax.experimental import pallas as pl
from jax.experimental.pallas import tpu as pltpu
from jax.experimental.pallas import tpu_sc as plsc

N = 10000
E = 320000
H = 128
NC = 2
NS = 16
NW = NC * NS
K = 128
EPT = 10240
BATCHES = EPT // K
HALF_B = BATCHES // 2
E_PAD = NW * EPT
GARBAGE_ROW = N
AI_ROWS = N + 16
RPT = 624


def _mm_body(x_ref, wt_ref, b_ref, o_ref):
    acc = jnp.dot(x_ref[...], wt_ref[...], preferred_element_type=jnp.float32)
    o_ref[...] = jnp.maximum(acc + b_ref[...], 0.0)


def _matmul_relu(x2, wt, b2):
    return pl.pallas_call(
        _mm_body,
        grid=(20,),
        in_specs=[
            pl.BlockSpec((1000, H), lambda i: (i, 0)),
            pl.BlockSpec((H, H), lambda i: (0, 0)),
            pl.BlockSpec((1, H), lambda i: (0, 0)),
        ],
        out_specs=pl.BlockSpec((1000, H), lambda i: (i, 0)),
        out_shape=jax.ShapeDtypeStruct((2 * N, H), jnp.float32),
    )(x2, wt, b2)


def _sc_body(idx_hbm, table_hbm, out_hbm,
             stage, g0, g1, ai_sh, semg0, semg1):
    cid = lax.axis_index("c")
    sid = lax.axis_index("s")
    wid = cid * NS + sid

    zero16 = jnp.zeros((16,), jnp.float32)

    def _zrow(r, c):
        for j in range(H // 16):
            g0[r, pl.ds(j * 16, 16)] = zero16
        return c

    lax.fori_loop(0, K, _zrow, 0)
    for k in range(RPT // K):
        pltpu.sync_copy(g0, ai_sh.at[pl.ds(sid * RPT + k * K, K)])
    _rem = RPT - (RPT // K) * K
    pltpu.sync_copy(g0.at[pl.ds(0, _rem)],
                    ai_sh.at[pl.ds(sid * RPT + (RPT // K) * K, _rem)])

    @pl.when(sid == NS - 1)
    def _zero_tail():
        pltpu.sync_copy(g0.at[pl.ds(0, AI_ROWS - NS * RPT)],
                        ai_sh.at[pl.ds(NS * RPT, AI_ROWS - NS * RPT)])

    plsc.subcore_barrier()

    def _gather(jt, g, sem):
        pltpu.async_copy(table_hbm.at[stage.at[jt, 0]], g, sem)

    def _wait_g(g, sem):
        pltpu.make_async_copy(table_hbm.at[stage.at[0, 0]], g, sem).wait()

    def _scatter(jt, g):
        pltpu.sync_copy(g, ai_sh.at[stage.at[jt, 1]], add=True)

    for h in range(2):
        pltpu.sync_copy(idx_hbm.at[wid, h], stage)
        _gather(0, g0, semg0)

        def _edge_body(t, c):
            j = 2 * t
            _gather(j + 1, g1, semg1)
            _wait_g(g0, semg0)
            _scatter(j, g0)
            _gather(j + 2, g0, semg0)
            _wait_g(g1, semg1)
            _scatter(j + 1, g1)
            return c

        lax.fori_loop(0, HALF_B // 2 - 1, _edge_body, 0)
        _gather(HALF_B - 1, g1, semg1)
        _wait_g(g0, semg0)
        _scatter(HALF_B - 2, g0)
        _wait_g(g1, semg1)
        _scatter(HALF_B - 1, g1)

    plsc.subcore_barrier()
    pltpu.sync_copy(ai_sh.at[pl.ds(sid * RPT, RPT)],
                    out_hbm.at[pl.ds(cid * N + sid * RPT, RPT)])

    @pl.when(sid == NS - 1)
    def _write_tail():
        pltpu.sync_copy(ai_sh.at[pl.ds(NS * RPT, N - NS * RPT)],
                        out_hbm.at[pl.ds(cid * N + NS * RPT, N - NS * RPT)])


@functools.cache
def _sc_scatter():
    return pl.kernel(
        _sc_body,
        out_type=jax.ShapeDtypeStruct((NC * N, H), jnp.float32),
        mesh=plsc.VectorSubcoreMesh(core_axis_name="c", subcore_axis_name="s"),
        scratch_types=[
            pltpu.VMEM((HALF_B, 2, K), jnp.int32),
            pltpu.VMEM((K, H), jnp.float32),
            pltpu.VMEM((K, H), jnp.float32),
            pltpu.VMEM_SHARED((AI_ROWS, H), jnp.float32),
            pltpu.SemaphoreType.DMA,
            pltpu.SemaphoreType.DMA,
        ],
    )


def _fin_body(sir_ref, ai_ref, x4_ref, lnw_ref, lnb_ref, o_ref):
    s = sir_ref[0]
    i = sir_ref[1]
    ai = ai_ref[0] + ai_ref[1]
    x4 = x4_ref[...]
    beta = x4[:, 0:1]
    gamma = x4[:, 1:2]
    ds = -beta * (ai * s)
    di = -ds - gamma * i
    dr = gamma * i
    w = lnw_ref[...]
    b = lnb_ref[...]

    def _ln(v):
        m = jnp.mean(v, axis=-1, keepdims=True)
        cvar = v - m
        var = jnp.mean(cvar * cvar, axis=-1, keepdims=True)
        return cvar * lax.rsqrt(var + 1e-5) * w + b

    o_ref[0] = _ln(ds)
    o_ref[1] = _ln(di)
    o_ref[2] = _ln(dr)
    o_ref[3] = x4


def _finalize(sir2, ai2, x4, lnw2, lnb2):
    return pl.pallas_call(
        _fin_body,
        grid=(10,),
        in_specs=[
            pl.BlockSpec((2, 1000, H), lambda j: (0, j, 0)),
            pl.BlockSpec((2, 1000, H), lambda j: (0, j, 0)),
            pl.BlockSpec((1000, H), lambda j: (j, 0)),
            pl.BlockSpec((1, H), lambda j: (0, 0)),
            pl.BlockSpec((1, H), lambda j: (0, 0)),
        ],
        out_specs=pl.BlockSpec((4, 1000, H), lambda j: (0, j, 0)),
        out_shape=jax.ShapeDtypeStruct((4, N, H), jnp.float32),
    )(sir2, ai2, x4, lnw2, lnb2)


def kernel(t, x, edge_index, W, b, ln_w, ln_b):
    del t
    x2 = x[: 2 * N]
    x4 = x[3 * N:]

    sir = _matmul_relu(x2, W.T, b.reshape(1, H))

    n_pad = E_PAD - E
    rows = jnp.concatenate(
        [edge_index[0], jnp.full((n_pad,), GARBAGE_ROW, jnp.int32)])
    cols = jnp.concatenate(
        [edge_index[1] + jnp.int32(N), jnp.zeros((n_pad,), jnp.int32)])
    idx = jnp.stack([cols.reshape(NW, BATCHES, K),
                     rows.reshape(NW, BATCHES, K)], axis=2)
    idx = idx.reshape(NW, 2, HALF_B, 2, K)

    ai_partials = _sc_scatter()(idx, sir)

    out = _finalize(sir.reshape(2, N, H), ai_partials.reshape(2, N, H),
                    x4, ln_w.reshape(1, H), ln_b.reshape(1, H))
    return out.reshape(4 * N, H)
